# baseline (speedup 1.0000x reference)
import functools

import jax
import jax.numpy as jnp
from jax import lax
from jax.experimental import pallas as pl
from jax.experimental.pallas import tpu as pltpu
from jax.experimental.pallas import tpu_sc as plsc

BATCH = 16384
N_HID = 64
N_TAB = 4
LANES = 16
NUM_CORES = 2
NUM_SUBCORES = 16
NW = NUM_CORES * NUM_SUBCORES
BPW = BATCH // NW
CHUNK = 128
NCHUNK = BPW // CHUNK

_mesh = plsc.VectorSubcoreMesh(core_axis_name="c", subcore_axis_name="s")


@functools.partial(
    pl.kernel,
    mesh=_mesh,
    out_type=jax.ShapeDtypeStruct((BATCH, N_HID), jnp.float32),
    compiler_params=pltpu.CompilerParams(use_tc_tiling_on_sc=False),
    scratch_types=[
        pltpu.VMEM((N_TAB, BPW), jnp.int32),
        pltpu.VMEM((N_TAB, CHUNK, N_HID), jnp.float32),
        pltpu.VMEM((CHUNK, N_HID), jnp.float32),
        pltpu.SemaphoreType.DMA,
    ],
)
def _lookup_sum(tT, e0, e1, e2, e3, out, idx_v, bufs, obuf, sem):
    wid = lax.axis_index("s") * NUM_CORES + lax.axis_index("c")
    base = wid * BPW
    tabs = (e0, e1, e2, e3)

    for k in range(N_TAB):
        pltpu.sync_copy(tT.at[k, pl.ds(base, BPW)], idx_v.at[k])

    for c in range(NCHUNK):
        off = c * CHUNK
        cps = [
            pltpu.async_copy(tabs[k].at[idx_v.at[k, pl.ds(off, CHUNK)]],
                             bufs.at[k], sem)
            for k in range(N_TAB)
        ]
        for cp in cps:
            cp.wait()

        def row_body(r, _):
            for j in range(N_HID // LANES):
                o = j * LANES
                v = (bufs[0, r, pl.ds(o, LANES)]
                     + bufs[1, r, pl.ds(o, LANES)]
                     + bufs[2, r, pl.ds(o, LANES)]
                     + bufs[3, r, pl.ds(o, LANES)])
                obuf[r, pl.ds(o, LANES)] = v
            return 0

        lax.fori_loop(0, CHUNK, row_body, 0)
        pltpu.sync_copy(obuf, out.at[pl.ds(base + off, CHUNK)])


def kernel(t, emb0, emb1, emb2, emb3):
    tT = t.T.reshape(N_TAB, BATCH)
    return _lookup_sum(tT, emb0, emb1, emb2, emb3)


# speedup vs baseline: 1.3330x; 1.3330x over previous
import functools

import jax
import jax.numpy as jnp
from jax import lax
from jax.experimental import pallas as pl
from jax.experimental.pallas import tpu as pltpu
from jax.experimental.pallas import tpu_sc as plsc

BATCH = 16384
N_HID = 64
N_TAB = 4
LANES = 16
NUM_CORES = 2
NUM_SUBCORES = 16
NW = NUM_CORES * NUM_SUBCORES
BPW = BATCH // NW
CHUNK = 64
NCHUNK = BPW // CHUNK
NGRP = CHUNK // LANES

_mesh = plsc.VectorSubcoreMesh(core_axis_name="c", subcore_axis_name="s")


@functools.partial(
    pl.kernel,
    mesh=_mesh,
    out_type=jax.ShapeDtypeStruct((BATCH, N_HID), jnp.float32),
    scratch_types=[
        pltpu.VMEM((N_TAB * BPW,), jnp.int32),
        pltpu.VMEM((N_TAB, CHUNK, N_HID), jnp.float32),
        pltpu.SemaphoreType.DMA,
    ],
)
def _lookup_sum(tT, e0, e1, e2, e3, out, tvv, bufs, sem):
    wid = lax.axis_index("s") * NUM_CORES + lax.axis_index("c")
    base = wid * BPW
    tabs = (e0, e1, e2, e3)

    for k in range(N_TAB):
        pltpu.sync_copy(tT.at[k, pl.ds(base, BPW)],
                        tvv.at[pl.ds(k * BPW, BPW)])

    for c in range(NCHUNK):
        gb = base + c * CHUNK

        def fire(g, _):
            row0 = g * LANES
            for k in range(N_TAB):
                iv = tvv[pl.ds(k * BPW + c * CHUNK + row0, LANES)]
                for j in range(LANES):
                    pltpu.async_copy(tabs[k].at[iv[j]],
                                     bufs.at[k, row0 + j], sem)
            return 0

        lax.fori_loop(0, NGRP, fire, 0)

        for k in range(N_TAB):
            pltpu.make_async_copy(tabs[k].at[pl.ds(0, CHUNK), :],
                                  bufs.at[k], sem).wait()

        def srow(r, _):
            for j in range(N_HID // LANES):
                o = j * LANES
                v = (bufs[0, r, pl.ds(o, LANES)]
                     + bufs[1, r, pl.ds(o, LANES)]
                     + bufs[2, r, pl.ds(o, LANES)]
                     + bufs[3, r, pl.ds(o, LANES)])
                bufs[0, r, pl.ds(o, LANES)] = v
            return 0

        lax.fori_loop(0, CHUNK, srow, 0)
        pltpu.sync_copy(bufs.at[0], out.at[pl.ds(gb, CHUNK), :])


def kernel(t, emb0, emb1, emb2, emb3):
    tT = t.T.reshape(N_TAB, BATCH)
    return _lookup_sum(tT, emb0, emb1, emb2, emb3)
